# baseline (speedup 1.0000x reference)
import functools

import jax
import jax.numpy as jnp
from jax.experimental import pallas as pl
from jax.experimental.pallas import tpu as pltpu

_N = 500000
_C = 100
_NBINS = 10
_BLK = 4000


def _calib_kernel(bounds_ref, x_ref, tgt_ref, out_ref, cnt_ref, acc_ref, *, nsteps):
    step = pl.program_id(0)

    @pl.when(step == 0)
    def _init():
        cnt_ref[...] = jnp.zeros_like(cnt_ref)
        acc_ref[...] = jnp.zeros_like(acc_ref)

    x = x_ref[...]
    rowmax = jnp.max(x, axis=1, keepdims=True)
    e = jnp.exp(x - rowmax)
    s = jnp.sum(e, axis=1, keepdims=True)
    conf = 1.0 / s

    iota = jax.lax.broadcasted_iota(jnp.int32, x.shape, 1)
    pred = jnp.min(jnp.where(x == rowmax, iota, _C), axis=1, keepdims=True)
    accv = (pred == tgt_ref[...]).astype(jnp.float32)

    lower = bounds_ref[0:1, :]
    upper = bounds_ref[1:2, :]
    inside = (conf > lower) & (conf <= upper)
    cnt_ref[0:1, :] += jnp.sum(inside.astype(jnp.float32), axis=0, keepdims=True)
    acc_ref[0:1, :] += jnp.sum(jnp.where(inside, accv, 0.0), axis=0, keepdims=True)

    @pl.when(step == nsteps - 1)
    def _fin():
        c = cnt_ref[...]
        a = acc_ref[...]
        out_ref[...] = jnp.where(c > 0, a / jnp.maximum(c, 1.0), 0.0)


@jax.jit
def kernel(logits, targets):
    interval = jnp.linspace(0.0, 1.0, _NBINS + 1)
    lower = jnp.concatenate([interval[:-1], jnp.full((128 - _NBINS,), 2.0)])
    upper = jnp.concatenate([interval[1:], jnp.full((128 - _NBINS,), 3.0)])
    bounds = jnp.zeros((8, 128), jnp.float32)
    bounds = bounds.at[0, :].set(lower).at[1, :].set(upper)

    tgt = targets.astype(jnp.int32).reshape(_N, 1)
    nsteps = _N // _BLK

    out = pl.pallas_call(
        functools.partial(_calib_kernel, nsteps=nsteps),
        grid=(nsteps,),
        in_specs=[
            pl.BlockSpec((8, 128), lambda i: (0, 0)),
            pl.BlockSpec((_BLK, _C), lambda i: (i, 0)),
            pl.BlockSpec((_BLK, 1), lambda i: (i, 0)),
        ],
        out_specs=pl.BlockSpec((8, 128), lambda i: (0, 0)),
        out_shape=jax.ShapeDtypeStruct((8, 128), jnp.float32),
        scratch_shapes=[
            pltpu.VMEM((8, 128), jnp.float32),
            pltpu.VMEM((8, 128), jnp.float32),
        ],
    )(bounds, logits, tgt)

    return out[0, :_NBINS].reshape(_NBINS, 1)
